# baseline (speedup 1.0000x reference)
import jax
import jax.numpy as jnp
from jax.experimental import pallas as pl
from jax.experimental.pallas import tpu as pltpu

_CHUNK_ROWS = 2048


def _make_kernel(bs, seq_l, d):
    n_chunks = seq_l // _CHUNK_ROWS

    def body(t_hbm, o_hbm, buf, rsems, wsem):
        reads = []
        for c in range(n_chunks):
            sl = pl.ds(c * _CHUNK_ROWS, _CHUNK_ROWS)
            cp = pltpu.make_async_copy(t_hbm.at[sl], buf.at[sl], rsems.at[c])
            cp.start()
            reads.append(cp)
        writes = []
        for c in range(n_chunks):
            reads[c].wait()
            sl = pl.ds(c * _CHUNK_ROWS, _CHUNK_ROWS)
            for b in range(bs):
                w = pltpu.make_async_copy(buf.at[sl], o_hbm.at[b, sl], wsem)
                w.start()
                writes.append(w)
        for w in writes:
            w.wait()

    return pl.pallas_call(
        body,
        in_specs=[pl.BlockSpec(memory_space=pltpu.MemorySpace.HBM)],
        out_specs=pl.BlockSpec(memory_space=pltpu.MemorySpace.HBM),
        out_shape=jax.ShapeDtypeStruct((bs, seq_l, d), jnp.float32),
        scratch_shapes=[
            pltpu.VMEM((seq_l, d), jnp.float32),
            pltpu.SemaphoreType.DMA((n_chunks,)),
            pltpu.SemaphoreType.DMA,
        ],
    )


def kernel(x, table):
    bs, seq_l, d = x.shape
    return _make_kernel(bs, seq_l, d)(table)


# speedup vs baseline: 1.0460x; 1.0460x over previous
import jax
import jax.numpy as jnp
from jax.experimental import pallas as pl
from jax.experimental.pallas import tpu as pltpu


def _chunk_plan(seq_l):
    plan = []
    remaining = seq_l
    size = 128
    while remaining > 0:
        size = min(size, remaining)
        plan.append(size)
        remaining -= size
        if size < 1024 and len(plan) % 2 == 0:
            size *= 2
    return plan


def _make_kernel(bs, seq_l, d):
    chunks = _chunk_plan(seq_l)
    offs = [sum(chunks[:i]) for i in range(len(chunks))]

    def body(t_hbm, o_hbm, buf, rsems, wsem):
        reads = []
        for c, (o, n) in enumerate(zip(offs, chunks)):
            sl = pl.ds(o, n)
            cp = pltpu.make_async_copy(t_hbm.at[sl], buf.at[sl], rsems.at[c])
            cp.start()
            reads.append(cp)
        writes = []
        for c, (o, n) in enumerate(zip(offs, chunks)):
            reads[c].wait()
            sl = pl.ds(o, n)
            for b in range(bs):
                w = pltpu.make_async_copy(buf.at[sl], o_hbm.at[b, sl], wsem)
                w.start()
                writes.append(w)
        for w in writes:
            w.wait()

    return pl.pallas_call(
        body,
        in_specs=[pl.BlockSpec(memory_space=pltpu.MemorySpace.HBM)],
        out_specs=pl.BlockSpec(memory_space=pltpu.MemorySpace.HBM),
        out_shape=jax.ShapeDtypeStruct((bs, seq_l, d), jnp.float32),
        scratch_shapes=[
            pltpu.VMEM((seq_l, d), jnp.float32),
            pltpu.SemaphoreType.DMA((len(chunks),)),
            pltpu.SemaphoreType.DMA,
        ],
    )


def kernel(x, table):
    bs, seq_l, d = x.shape
    return _make_kernel(bs, seq_l, d)(table)
